# baseline (speedup 1.0000x reference)
import functools

import jax
import jax.numpy as jnp
from jax import lax
from jax.experimental import pallas as pl
from jax.experimental.pallas import tpu as pltpu
from jax.experimental.pallas import tpu_sc as plsc

VOCAB = 1000000
EMBED = 64
BATCH = 16384

NC = 2
NS = 16
NW = NC * NS

NTILE = (VOCAB + 127) // 128
TC_BASE = NTILE // NW
TC_EXTRA = NTILE - TC_BASE * NW
LAST_LEN = VOCAB - (NTILE - 1) * 128

SEG = 640
IDX_PARTS = 8
IDX_CHUNK = BATCH // IDX_PARTS


def _sc_body(idx_hbm, tbl_t_hbm, tail_hbm, out_hbm,
             idxa, lidx, lpos, pos2d, buf0, buf1, buf2, buft, rows_a,
             rows_b, rowsw, sem0, sem1, sem2, semsc):
    wid = lax.axis_index("s") * NC + lax.axis_index("c")
    tc0 = wid * TC_BASE + jnp.minimum(wid, TC_EXTRA)
    ntc = TC_BASE + jnp.where(wid < TC_EXTRA, 1, 0)
    vlo = tc0 * 128
    vhi = jnp.minimum(vlo + ntc * 128, VOCAB)
    has_tail = vlo + ntc * 128 > VOCAB
    ntc_full = ntc - has_tail.astype(jnp.int32)

    iota16 = lax.broadcasted_iota(jnp.int32, (16,), 0)
    kg = [iota16 + 16 * g for g in range(4)]

    def count_body(v, tot):
        iv = idxa[pl.ds(v * 16, 16)]
        m = (iv >= vlo) & (iv < vhi)
        return tot + plsc.all_reduce_population_count(m)[0]

    total = jnp.int32(0)
    for part in range(IDX_PARTS):
        pltpu.sync_copy(idx_hbm.at[pl.ds(part * IDX_CHUNK, IDX_CHUNK)], idxa)
        total = lax.fori_loop(0, IDX_CHUNK // 16, count_body, total)
    nseg = jnp.maximum(jnp.int32(1), (total + SEG - 1) // SEG)

    def seg_body(s, _unused):
        skip = s * SEG

        def make_filt_body(pbase):
            def filt_body(v, carry):
                seen, nk = carry
                iv = idxa[pl.ds(v * 16, 16)]
                pv = iota16 + pbase + v * 16
                m = (iv >= vlo) & (iv < vhi)
                rank = plsc.cumsum(m.astype(jnp.int32)) + seen
                keep = m & (rank > skip) & (rank <= skip + SEG)
                plsc.store_compressed(lidx.at[pl.ds(nk, 16)], iv, mask=keep)
                plsc.store_compressed(lpos.at[pl.ds(nk, 16)], pv, mask=keep)
                seen = seen + plsc.all_reduce_population_count(m)[0]
                nk = nk + plsc.all_reduce_population_count(keep)[0]
                return seen, nk

            return filt_body

        carry = (jnp.int32(0), jnp.int32(0))
        for part in range(IDX_PARTS):
            pltpu.sync_copy(idx_hbm.at[pl.ds(part * IDX_CHUNK, IDX_CHUNK)], idxa)
            carry = lax.fori_loop(0, IDX_CHUNK // 16,
                                  make_filt_body(part * IDX_CHUNK), carry)
        _, nloc = carry
        nlv = (nloc + 15) // 16

        bufs = (buf0, buf1, buf2)
        sems = (sem0, sem1, sem2)

        def chunk_copy(c, q):
            cb = vlo + c * 128
            pltpu.make_async_copy(
                tbl_t_hbm.at[:, pl.ds(cb, 128)], bufs[q], sems[q]).start()

        def chunk_wait(c, q):
            cb = vlo + c * 128
            pltpu.make_async_copy(
                tbl_t_hbm.at[:, pl.ds(cb, 128)], bufs[q], sems[q]).wait()

        def process_at(cb, buf):

            def vreg_body(sg, _c2):
                sidx = lidx[pl.ds(sg * 16, 16)]
                m = (sidx >= cb) & (sidx < cb + 128)
                c0 = plsc.all_reduce_population_count(m)[0]

                @pl.when(c0 > 0)
                def _():
                    mi = m.astype(jnp.int32)
                    for t in range(16):
                        @pl.when((mi[t] > 0) & (sg * 16 + t < nloc))
                        def _():
                            col = sidx[t] - cb
                            slot = sg * 16 + t
                            for g in range(4):
                                vals = plsc.load_gather(
                                    buf, [kg[g], jnp.full((16,), col, jnp.int32)])

                                @pl.when(slot < 512)
                                def _(g=g, vals=vals):
                                    rows_a[slot, pl.ds(16 * g, 16)] = vals

                                @pl.when(slot >= 512)
                                def _(g=g, vals=vals):
                                    rows_b[slot - 512, pl.ds(16 * g, 16)] = vals
                return _c2

            lax.fori_loop(0, nlv, vreg_body, jnp.int32(0))

        for q in range(2):
            @pl.when(q < ntc_full)
            def _(q=q):
                chunk_copy(jnp.int32(q), q)

        ntri = (ntc_full + 2) // 3

        def tri_body(t, _unused2):
            for q in range(3):
                c = 3 * t + q

                @pl.when(c < ntc_full)
                def _(c=c, q=q):
                    @pl.when(c + 2 < ntc_full)
                    def _():
                        chunk_copy(c + 2, (q + 2) % 3)

                    chunk_wait(c, q)
                    process_at(vlo + c * 128, bufs[q])
            return _unused2

        lax.fori_loop(0, ntri, tri_body, jnp.int32(0))

        @pl.when(has_tail)
        def _():
            pltpu.sync_copy(tail_hbm, buft)
            process_at(jnp.int32((NTILE - 1) * 128), buft)

        @pl.when(nloc > 0)
        def _():
            lastp = jnp.full((16,), lpos[pl.ds(nloc - 1, 16)][0], jnp.int32)
            li = nloc - 1
            lastrow = [
                jnp.where(
                    li < 512,
                    rows_a[jnp.minimum(li, 511), pl.ds(16 * g, 16)],
                    rows_b[jnp.maximum(li - 512, 0), pl.ds(16 * g, 16)],
                )
                for g in range(4)
            ]

            def pad_body(p, _u):
                @pl.when(p >= nloc)
                def _():
                    lpos[pl.ds(p, 16)] = lastp
                    for g in range(4):
                        @pl.when(p < 512)
                        def _(g=g):
                            rows_a[p, pl.ds(16 * g, 16)] = lastrow[g]

                        @pl.when(p >= 512)
                        def _(g=g):
                            rows_b[p - 512, pl.ds(16 * g, 16)] = lastrow[g]
                return _u

            ngrp_pad = ((nloc + 63) // 64) * 64
            lax.fori_loop(nloc, ngrp_pad, pad_body, jnp.int32(0))

            for r in range(SEG // 64):
                for g in range(4):
                    pos2d[r, pl.ds(16 * g, 16)] = lpos[pl.ds(r * 64 + g * 16, 16)]

            for r in range(SEG // 64):
                @pl.when(r * 64 < nloc)
                def _(r=r):
                    src = rows_a if r < 8 else rows_b
                    base = r * 64 if r < 8 else (r - 8) * 64

                    def widen_body(i, _w):
                        for g in range(4):
                            rowsw[i, pl.ds(16 * g, 16)] = (
                                src[base + i, pl.ds(16 * g, 16)])
                        return _w

                    lax.fori_loop(0, 64, widen_body, jnp.int32(0))
                    pltpu.make_async_copy(
                        rowsw, out_hbm.at[pos2d.at[r]], semsc).start()
                    pltpu.make_async_copy(
                        rowsw, out_hbm.at[pos2d.at[r]], semsc).wait()

        return _unused

    lax.fori_loop(0, nseg, seg_body, jnp.int32(0))


def _sc_stream_gather(idx, tbl_t, tail_t):
    mesh = plsc.VectorSubcoreMesh(
        core_axis_name="c", subcore_axis_name="s", num_cores=NC, num_subcores=NS
    )
    k = pl.kernel(
        _sc_body,
        out_type=jax.ShapeDtypeStruct((BATCH, 128), jnp.float32),
        mesh=mesh,
        scratch_types=[
            pltpu.VMEM((IDX_CHUNK,), jnp.int32),
            pltpu.VMEM((SEG + 16,), jnp.int32),
            pltpu.VMEM((SEG + 16,), jnp.int32),
            pltpu.VMEM((SEG // 64, 64), jnp.int32),
            pltpu.VMEM((EMBED, 128), jnp.float32),
            pltpu.VMEM((EMBED, 128), jnp.float32),
            pltpu.VMEM((EMBED, 128), jnp.float32),
            pltpu.VMEM((EMBED, LAST_LEN), jnp.float32),
            pltpu.VMEM((512, EMBED), jnp.float32),
            pltpu.VMEM((128, EMBED), jnp.float32),
            pltpu.VMEM((64, 128), jnp.float32),
            pltpu.SemaphoreType.DMA,
            pltpu.SemaphoreType.DMA,
            pltpu.SemaphoreType.DMA,
            pltpu.SemaphoreType.DMA,
        ],
        compiler_params=pltpu.CompilerParams(
            use_tc_tiling_on_sc=True, needs_layout_passes=False
        ),
    )
    return k(idx, tbl_t, tail_t)


ROWS_BLK = 2048


def _tc_body(x_ref, w_ref, b_ref, g_ref, bt_ref, o_ref):
    x = x_ref[:, :EMBED]
    w = w_ref[...]
    y = lax.dot_general(
        x, w, (((1,), (1,)), ((), ())), preferred_element_type=jnp.float32
    )
    y = y + b_ref[...]
    mean = jnp.mean(y, axis=-1, keepdims=True)
    var = jnp.mean((y - mean) ** 2, axis=-1, keepdims=True)
    xn = (y - mean) * lax.rsqrt(var + 1e-5)
    o_ref[...] = xn * g_ref[...] + bt_ref[...]


def _linear_ln(x, W, b, gamma, beta):
    grid = BATCH // ROWS_BLK
    return pl.pallas_call(
        _tc_body,
        out_shape=jax.ShapeDtypeStruct((BATCH, EMBED), jnp.float32),
        grid=(grid,),
        in_specs=[
            pl.BlockSpec((ROWS_BLK, 128), lambda i: (i, 0)),
            pl.BlockSpec((EMBED, EMBED), lambda i: (0, 0)),
            pl.BlockSpec((1, EMBED), lambda i: (0, 0)),
            pl.BlockSpec((1, EMBED), lambda i: (0, 0)),
            pl.BlockSpec((1, EMBED), lambda i: (0, 0)),
        ],
        out_specs=pl.BlockSpec((ROWS_BLK, EMBED), lambda i: (i, 0)),
    )(x, W, b.reshape(1, EMBED), gamma.reshape(1, EMBED), beta.reshape(1, EMBED))


def kernel(x_idx, emb_table, W, b, gamma, beta):
    idx = x_idx.astype(jnp.int32)
    tbl_t = emb_table.T
    g = _sc_stream_gather(idx, tbl_t, tbl_t[:, (NTILE - 1) * 128:])
    return _linear_ln(g, W, b, gamma, beta)


# speedup vs baseline: 1.8650x; 1.8650x over previous
import functools

import jax
import jax.numpy as jnp
from jax import lax
from jax.experimental import pallas as pl
from jax.experimental.pallas import tpu as pltpu
from jax.experimental.pallas import tpu_sc as plsc

VOCAB = 1000000
EMBED = 64
BATCH = 16384

NC = 2
NS = 16
NW = NC * NS

NTILE = (VOCAB + 127) // 128
TC_BASE = NTILE // NW
TC_EXTRA = NTILE - TC_BASE * NW
LAST_LEN = VOCAB - (NTILE - 1) * 128

SEG = 640
NVREG = BATCH // 16
NBUF = 3


def _sc_body(idx_hbm, tbl_t_hbm, tail_hbm, out_hbm,
             idxa, lidx, lpos, pos2d, buf0, buf1, buf2, buft, rows,
             sem, sem2):
    wid = lax.axis_index("s") * NC + lax.axis_index("c")
    tc0 = wid * TC_BASE + jnp.minimum(wid, TC_EXTRA)
    ntc = TC_BASE + jnp.where(wid < TC_EXTRA, 1, 0)
    vlo = tc0 * 128
    vhi = jnp.minimum(vlo + ntc * 128, VOCAB)
    has_tail = vlo + ntc * 128 > VOCAB
    ntc_full = ntc - has_tail.astype(jnp.int32)

    iota16 = lax.broadcasted_iota(jnp.int32, (16,), 0)
    kg = [iota16 + 16 * g for g in range(4)]

    def count_body(v, tot):
        iv = idxa[pl.ds(v * 16, 16)]
        m = (iv >= vlo) & (iv < vhi)
        return tot + plsc.all_reduce_population_count(m)[0]

    total = jnp.int32(0)
    for part in range(2):
        pltpu.sync_copy(idx_hbm.at[pl.ds(part * (BATCH // 2), BATCH // 2)], idxa)
        total = lax.fori_loop(0, NVREG // 2, count_body, total)
    nseg = jnp.maximum(jnp.int32(1), (total + SEG - 1) // SEG)

    def seg_body(s, _unused):
        skip = s * SEG

        def make_filt_body(pbase):
          def filt_body(v, carry):
            seen, nk = carry
            iv = idxa[pl.ds(v * 16, 16)]
            pv = iota16 + pbase + v * 16
            m = (iv >= vlo) & (iv < vhi)
            rank = plsc.cumsum(m.astype(jnp.int32)) + seen
            keep = m & (rank > skip) & (rank <= skip + SEG)
            plsc.store_compressed(lidx.at[pl.ds(nk, 16)], iv, mask=keep)
            plsc.store_compressed(lpos.at[pl.ds(nk, 16)], pv, mask=keep)
            seen = seen + plsc.all_reduce_population_count(m)[0]
            nk = nk + plsc.all_reduce_population_count(keep)[0]
            return seen, nk
          return filt_body

        carry = (jnp.int32(0), jnp.int32(0))
        for part in range(2):
            pltpu.sync_copy(
                idx_hbm.at[pl.ds(part * (BATCH // 2), BATCH // 2)], idxa)
            carry = lax.fori_loop(0, NVREG // 2,
                                  make_filt_body(part * (BATCH // 2)), carry)
        _, nloc = carry
        nlv = (nloc + 15) // 16

        bufs = (buf0, buf1, buf2)

        def chunk_copy(c, q):
            cb = vlo + c * 128
            pltpu.make_async_copy(
                tbl_t_hbm.at[:, pl.ds(cb, 128)], bufs[q], sem).start()

        def chunk_wait(c, q):
            cb = vlo + c * 128
            pltpu.make_async_copy(
                tbl_t_hbm.at[:, pl.ds(cb, 128)], bufs[q], sem).wait()

        def process_at(cb, buf):

            def vreg_body(sg, _c2):
                sidx = lidx[pl.ds(sg * 16, 16)]
                m = (sidx >= cb) & (sidx < cb + 128)
                c0 = plsc.all_reduce_population_count(m)[0]

                @pl.when(c0 > 0)
                def _():
                    mi = m.astype(jnp.int32)
                    for t in range(16):
                        @pl.when((mi[t] > 0) & (sg * 16 + t < nloc))
                        def _(t=t):
                            col = sidx[t] - cb
                            slot = sg * 16 + t
                            for g in range(4):
                                vals = plsc.load_gather(
                                    buf, [kg[g], jnp.full((16,), col, jnp.int32)])
                                rows[slot, pl.ds(16 * g, 16)] = vals
                return _c2

            lax.fori_loop(0, nlv, vreg_body, jnp.int32(0))

        for q in range(NBUF - 1):
            @pl.when(q < ntc_full)
            def _(q=q):
                chunk_copy(jnp.int32(q), q)

        nring = (ntc_full + NBUF - 1) // NBUF

        def ring_body(t, _unused2):
            for q in range(NBUF):
                c = NBUF * t + q

                @pl.when(c < ntc_full)
                def _(c=c, q=q):
                    @pl.when(c + NBUF - 1 < ntc_full)
                    def _():
                        chunk_copy(c + NBUF - 1, (q + NBUF - 1) % NBUF)

                    chunk_wait(c, q)
                    process_at(vlo + c * 128, bufs[q])
            return _unused2

        lax.fori_loop(0, nring, ring_body, jnp.int32(0))

        @pl.when(has_tail)
        def _():
            pltpu.sync_copy(tail_hbm, buft)
            process_at(jnp.int32((NTILE - 1) * 128), buft)

        @pl.when(nloc > 0)
        def _():
            lastp = jnp.full((16,), lpos[pl.ds(nloc - 1, 16)][0], jnp.int32)
            lastrow = [rows[nloc - 1, pl.ds(16 * g, 16)] for g in range(4)]

            def pad_body(p, _u):
                @pl.when(p >= nloc)
                def _():
                    lpos[pl.ds(p, 16)] = lastp
                    for g in range(4):
                        rows[p, pl.ds(16 * g, 16)] = lastrow[g]
                return _u

            ngrp_pad = ((nloc + 127) // 128) * 128
            lax.fori_loop(nloc, ngrp_pad, pad_body, jnp.int32(0))

            for r in range(SEG // 128):
                for g in range(8):
                    pos2d[r, pl.ds(16 * g, 16)] = lpos[pl.ds(r * 128 + g * 16, 16)]

            for r in range(SEG // 128):
                @pl.when(r * 128 < nloc)
                def _(r=r):
                    pltpu.make_async_copy(
                        rows.at[pl.ds(r * 128, 128)],
                        out_hbm.at[pos2d.at[r]],
                        sem2).start()
            for r in range(SEG // 128):
                @pl.when(r * 128 < nloc)
                def _(r=r):
                    pltpu.make_async_copy(
                        rows.at[pl.ds(r * 128, 128)],
                        out_hbm.at[pos2d.at[r]],
                        sem2).wait()

        return _unused

    lax.fori_loop(0, nseg, seg_body, jnp.int32(0))


def _sc_stream_gather(idx, tbl_t, tail_t):
    mesh = plsc.VectorSubcoreMesh(
        core_axis_name="c", subcore_axis_name="s", num_cores=NC, num_subcores=NS
    )
    k = pl.kernel(
        _sc_body,
        out_type=jax.ShapeDtypeStruct((BATCH, 128), jnp.float32),
        mesh=mesh,
        scratch_types=[
            pltpu.VMEM((BATCH // 2,), jnp.int32),
            pltpu.VMEM((SEG + 16,), jnp.int32),
            pltpu.VMEM((SEG + 16,), jnp.int32),
            pltpu.VMEM((SEG // 128, 128), jnp.int32),
            pltpu.VMEM((EMBED, 128), jnp.float32),
            pltpu.VMEM((EMBED, 128), jnp.float32),
            pltpu.VMEM((EMBED, 128), jnp.float32),
            pltpu.VMEM((EMBED, LAST_LEN), jnp.float32),
            pltpu.VMEM((SEG, 128), jnp.float32),
            pltpu.SemaphoreType.DMA,
            pltpu.SemaphoreType.DMA,
        ],
        compiler_params=pltpu.CompilerParams(
            use_tc_tiling_on_sc=True, needs_layout_passes=False
        ),
    )
    return k(idx, tbl_t, tail_t)


ROWS_BLK = 2048


def _tc_body(x_ref, w_ref, b_ref, g_ref, bt_ref, o_ref):
    x = x_ref[:, :EMBED]
    w = w_ref[...]
    y = lax.dot_general(
        x, w, (((1,), (1,)), ((), ())), preferred_element_type=jnp.float32
    )
    y = y + b_ref[...]
    mean = jnp.mean(y, axis=-1, keepdims=True)
    var = jnp.mean((y - mean) ** 2, axis=-1, keepdims=True)
    xn = (y - mean) * lax.rsqrt(var + 1e-5)
    o_ref[...] = xn * g_ref[...] + bt_ref[...]


def _linear_ln(x, W, b, gamma, beta):
    grid = BATCH // ROWS_BLK
    return pl.pallas_call(
        _tc_body,
        out_shape=jax.ShapeDtypeStruct((BATCH, EMBED), jnp.float32),
        grid=(grid,),
        in_specs=[
            pl.BlockSpec((ROWS_BLK, 128), lambda i: (i, 0)),
            pl.BlockSpec((EMBED, EMBED), lambda i: (0, 0)),
            pl.BlockSpec((1, EMBED), lambda i: (0, 0)),
            pl.BlockSpec((1, EMBED), lambda i: (0, 0)),
            pl.BlockSpec((1, EMBED), lambda i: (0, 0)),
        ],
        out_specs=pl.BlockSpec((ROWS_BLK, EMBED), lambda i: (i, 0)),
    )(x, W, b.reshape(1, EMBED), gamma.reshape(1, EMBED), beta.reshape(1, EMBED))


def kernel(x_idx, emb_table, W, b, gamma, beta):
    idx = x_idx.astype(jnp.int32)
    tbl_t = emb_table.T
    g = _sc_stream_gather(idx, tbl_t, tbl_t[:, (NTILE - 1) * 128:])
    return _linear_ln(g, W, b, gamma, beta)
